# baseline (speedup 1.0000x reference)
import functools

import jax
import jax.numpy as jnp
import numpy as np
from jax import lax
from jax.experimental import pallas as pl
from jax.experimental.pallas import tpu as pltpu
from jax.experimental.pallas import tpu_sc as plsc

_L = 128
_D = 32
_W = 256
_VP = 1024000


@functools.cache
def _build_tr(v):
    info = plsc.get_sparse_core_info()
    nw = info.num_cores * info.num_subcores
    vpw = v // nw
    n_u = vpw // _W

    mesh = plsc.VectorSubcoreMesh(core_axis_name="c", subcore_axis_name="s")

    @functools.partial(
        pl.kernel,
        out_type=jax.ShapeDtypeStruct((v, _D), jnp.float32),
        mesh=mesh,
        compiler_params=pltpu.CompilerParams(
            use_tc_tiling_on_sc=False, needs_layout_passes=False),
        scratch_types=[
            pltpu.VMEM((_D, _W), jnp.float32),
            pltpu.VMEM((_D, _W), jnp.float32),
            pltpu.VMEM((_W, _D + 1), jnp.float32),
            pltpu.VMEM((_W, _D + 1), jnp.float32),
            pltpu.SemaphoreType.DMA,
            pltpu.SemaphoreType.DMA,
            pltpu.SemaphoreType.DMA,
            pltpu.SemaphoreType.DMA,
        ],
    )
    def tr(wt_hbm, out_hbm, src0, src1, tb0, tb1, ssem0, ssem1, osem0, osem1):
        wid = lax.axis_index("s") * info.num_cores + lax.axis_index("c")
        v0 = wid * vpw
        ar = lax.iota(jnp.int32, 16)

        def fire(u, src, sem):
            pltpu.async_copy(wt_hbm.at[:, pl.ds(v0 + u * _W, _W)], src, sem)

        def sdrain(src, sem):
            pltpu.make_async_copy(wt_hbm.at[:, pl.ds(v0, _W)], src,
                                  sem).wait()

        def transpose(src, tb):
            for e in range(_D):
                ev = jnp.full((16,), e, jnp.int32)
                for g in range(_W // 16):
                    lv = g * 16 + ar
                    val = src[e, pl.ds(g * 16, 16)]
                    plsc.store_scatter(tb, [lv, ev], val)

        def outfire(u, tb, sem):
            pltpu.async_copy(tb.at[:, pl.ds(0, _D)],
                             out_hbm.at[pl.ds(v0 + u * _W, _W)], sem)

        def owait(tb, sem):
            pltpu.make_async_copy(tb.at[:, pl.ds(0, _D)],
                                  out_hbm.at[pl.ds(v0, _W)], sem).wait()

        def halfstep(u, src, tb, ssem, osem, src_nxt, ssem_nxt, prime, last):
            if not last:
                fire(u + 1, src_nxt, ssem_nxt)
            sdrain(src, ssem)
            if not prime:
                owait(tb, osem)
            transpose(src, tb)
            outfire(u, tb, osem)

        fire(0, src0, ssem0)
        halfstep(0, src0, tb0, ssem0, osem0, src1, ssem1, True, False)
        halfstep(1, src1, tb1, ssem1, osem1, src0, ssem0, True, False)

        def body(p, carry):
            a = 2 * p + 2
            halfstep(a, src0, tb0, ssem0, osem0, src1, ssem1, False, False)
            halfstep(a + 1, src1, tb1, ssem1, osem1, src0, ssem0,
                     False, False)
            return carry

        lax.fori_loop(0, (n_u - 3) // 2, body, 0)
        halfstep(n_u - 1, src0, tb0, ssem0, osem0, src1, ssem1, False, True)
        owait(tb0, osem0)
        owait(tb1, osem1)

    return tr


@functools.cache
def _build(b, h):
    info = plsc.get_sparse_core_info()
    nw = info.num_cores * info.num_subcores
    nbb = b // _L
    n_units = h * nbb
    upw = n_units // nw

    mesh = plsc.VectorSubcoreMesh(core_axis_name="c", subcore_axis_name="s")

    @functools.partial(
        pl.kernel,
        out_type=jax.ShapeDtypeStruct((h, _D // 8, nbb, 8, _L), jnp.float32),
        mesh=mesh,
        compiler_params=pltpu.CompilerParams(
            use_tc_tiling_on_sc=False, needs_layout_passes=False),
        scratch_types=[
            pltpu.VMEM((upw, _L), jnp.int32),
            pltpu.VMEM((_L, _D), jnp.float32),
            pltpu.VMEM((_L, _D), jnp.float32),
            pltpu.VMEM((_D // 8, 8, _L + 1), jnp.float32),
            pltpu.VMEM((_D // 8, 8, _L + 1), jnp.float32),
            pltpu.SemaphoreType.DMA,
            pltpu.SemaphoreType.DMA,
            pltpu.SemaphoreType.DMA,
            pltpu.SemaphoreType.DMA,
        ],
    )
    def emb(x_hbm, w_hbm, out_hbm, idx_v, rows0, rows1,
            tb0, tb1, gsem0, gsem1, osem0, osem1):
        wid = lax.axis_index("s") * info.num_cores + lax.axis_index("c")
        u0 = wid * upw
        pltpu.sync_copy(x_hbm.at[pl.ds(u0, upw)], idx_v)

        def fire(j, rows, sem):
            pltpu.async_copy(w_hbm.at[idx_v.at[j]], rows, sem)

        def drain(rows, sem):
            pltpu.make_async_copy(w_hbm.at[pl.ds(0, _L)], rows, sem).wait()

        ar = lax.iota(jnp.int32, 16)
        eg0 = lax.shift_right_logical(ar, 3)
        eg1 = eg0 + 2
        s_v = lax.bitwise_and(ar, 7)

        def transpose(rows, tb):
            for l in range(_L):
                lv = jnp.full((16,), l, jnp.int32)
                plsc.store_scatter(tb, [eg0, s_v, lv], rows[l, pl.ds(0, 16)])
                plsc.store_scatter(tb, [eg1, s_v, lv], rows[l, pl.ds(16, 16)])

        def outfire(j, tb, sem):
            u = u0 + j
            hh = u // nbb
            bb = u % nbb
            pltpu.async_copy(tb.at[:, :, pl.ds(0, _L)], out_hbm.at[hh, :, bb],
                             sem)

        def owait(tb, sem):
            pltpu.make_async_copy(tb.at[:, :, pl.ds(0, _L)],
                                  out_hbm.at[0, :, 0], sem).wait()

        def halfstep(j, rows, tb, gsem, osem, rows_nxt, gsem_nxt,
                     prime, last):
            if not last:
                fire(j + 1, rows_nxt, gsem_nxt)
            drain(rows, gsem)
            if not prime:
                owait(tb, osem)
            transpose(rows, tb)
            outfire(j, tb, osem)

        fire(0, rows0, gsem0)
        halfstep(0, rows0, tb0, gsem0, osem0, rows1, gsem1, True, False)
        halfstep(1, rows1, tb1, gsem1, osem1, rows0, gsem0, True, False)

        def body(u, carry):
            a = 2 * u
            halfstep(a, rows0, tb0, gsem0, osem0, rows1, gsem1, False, False)
            halfstep(a + 1, rows1, tb1, gsem1, osem1, rows0, gsem0,
                     False, False)
            return carry

        lax.fori_loop(1, upw // 2 - 1, body, 0)

        a = upw - 2
        halfstep(a, rows0, tb0, gsem0, osem0, rows1, gsem1, False, False)
        halfstep(a + 1, rows1, tb1, gsem1, osem1, rows0, gsem0, False, True)
        owait(tb0, osem0)
        owait(tb1, osem1)

    return emb


def kernel(x, weight):
    b, h = x.shape
    v, d = weight.shape
    xr = x.T.reshape(h * (b // _L), _L).astype(jnp.int32)
    wt = jnp.pad(weight.T, ((0, 0), (0, _VP - v)))
    wv = _build_tr(_VP)(wt)
    o5 = _build(b, h)(xr, wv)
    return o5.transpose(2, 4, 0, 1, 3).reshape(b, h, d)


# speedup vs baseline: 1.1259x; 1.1259x over previous
import functools

import jax
import jax.numpy as jnp
import numpy as np
from jax import lax
from jax.experimental import pallas as pl
from jax.experimental.pallas import tpu as pltpu
from jax.experimental.pallas import tpu_sc as plsc

_L = 128
_D = 32
_W = 256
_VP = 1024000


@functools.cache
def _build_tr(v):
    info = plsc.get_sparse_core_info()
    nw = info.num_cores * info.num_subcores
    nvb = v // _L
    vpw = v // nw
    n_u = vpw // _W
    uvb = _W // _L

    mesh = plsc.VectorSubcoreMesh(core_axis_name="c", subcore_axis_name="s")

    @functools.partial(
        pl.kernel,
        out_type=jax.ShapeDtypeStruct((v, _D), jnp.float32),
        mesh=mesh,
        compiler_params=pltpu.CompilerParams(
            use_tc_tiling_on_sc=False, needs_layout_passes=False),
        scratch_types=[
            pltpu.VMEM((_D // 8, uvb, 8, _L), jnp.float32),
            pltpu.VMEM((_D // 8, uvb, 8, _L), jnp.float32),
            pltpu.VMEM((_W, _D + 1), jnp.float32),
            pltpu.VMEM((_W, _D + 1), jnp.float32),
            pltpu.SemaphoreType.DMA,
            pltpu.SemaphoreType.DMA,
            pltpu.SemaphoreType.DMA,
            pltpu.SemaphoreType.DMA,
        ],
    )
    def tr(wt_hbm, out_hbm, src0, src1, tb0, tb1, ssem0, ssem1, osem0, osem1):
        wid = lax.axis_index("s") * info.num_cores + lax.axis_index("c")
        v0 = wid * vpw
        vb0 = wid * (vpw // _L)
        ar = lax.iota(jnp.int32, 16)

        def fire(u, src, sem):
            pltpu.async_copy(wt_hbm.at[:, pl.ds(vb0 + u * uvb, uvb)], src,
                             sem)

        def sdrain(src, sem):
            pltpu.make_async_copy(wt_hbm.at[:, pl.ds(0, uvb)], src,
                                  sem).wait()

        def transpose(src, tb):
            for eg in range(_D // 8):
                for s in range(8):
                    ev = jnp.full((16,), eg * 8 + s, jnp.int32)
                    for vbi in range(uvb):
                        for g in range(_L // 16):
                            lv = vbi * _L + g * 16 + ar
                            val = src[eg, vbi, s, pl.ds(g * 16, 16)]
                            plsc.store_scatter(tb, [lv, ev], val)

        def outfire(u, tb, sem):
            pltpu.async_copy(tb.at[:, pl.ds(0, _D)],
                             out_hbm.at[pl.ds(v0 + u * _W, _W)], sem)

        def owait(tb, sem):
            pltpu.make_async_copy(tb.at[:, pl.ds(0, _D)],
                                  out_hbm.at[pl.ds(v0, _W)], sem).wait()

        def halfstep(u, src, tb, ssem, osem, src_nxt, ssem_nxt, prime, last):
            if not last:
                fire(u + 1, src_nxt, ssem_nxt)
            sdrain(src, ssem)
            if not prime:
                owait(tb, osem)
            transpose(src, tb)
            outfire(u, tb, osem)

        fire(0, src0, ssem0)
        halfstep(0, src0, tb0, ssem0, osem0, src1, ssem1, True, False)
        halfstep(1, src1, tb1, ssem1, osem1, src0, ssem0, True, False)

        def body(p, carry):
            a = 2 * p + 2
            halfstep(a, src0, tb0, ssem0, osem0, src1, ssem1, False, False)
            halfstep(a + 1, src1, tb1, ssem1, osem1, src0, ssem0,
                     False, False)
            return carry

        lax.fori_loop(0, (n_u - 3) // 2, body, 0)
        halfstep(n_u - 1, src0, tb0, ssem0, osem0, src1, ssem1, False, True)
        owait(tb0, osem0)
        owait(tb1, osem1)

    return tr


@functools.cache
def _build(b, h):
    info = plsc.get_sparse_core_info()
    nw = info.num_cores * info.num_subcores
    nbb = b // _L
    n_units = h * nbb
    upw = n_units // nw

    mesh = plsc.VectorSubcoreMesh(core_axis_name="c", subcore_axis_name="s")

    @functools.partial(
        pl.kernel,
        out_type=jax.ShapeDtypeStruct((h, _D // 8, nbb, 8, _L), jnp.float32),
        mesh=mesh,
        compiler_params=pltpu.CompilerParams(
            use_tc_tiling_on_sc=False, needs_layout_passes=False),
        scratch_types=[
            pltpu.VMEM((upw, _L), jnp.int32),
            pltpu.VMEM((_L, _D), jnp.float32),
            pltpu.VMEM((_L, _D), jnp.float32),
            pltpu.VMEM((_D // 8, 8, _L + 1), jnp.float32),
            pltpu.VMEM((_D // 8, 8, _L + 1), jnp.float32),
            pltpu.SemaphoreType.DMA,
            pltpu.SemaphoreType.DMA,
            pltpu.SemaphoreType.DMA,
            pltpu.SemaphoreType.DMA,
        ],
    )
    def emb(x_hbm, w_hbm, out_hbm, idx_v, rows0, rows1,
            tb0, tb1, gsem0, gsem1, osem0, osem1):
        wid = lax.axis_index("s") * info.num_cores + lax.axis_index("c")
        u0 = wid * upw
        pltpu.sync_copy(x_hbm.at[pl.ds(u0, upw)], idx_v)

        def fire(j, rows, sem):
            pltpu.async_copy(w_hbm.at[idx_v.at[j]], rows, sem)

        def drain(rows, sem):
            pltpu.make_async_copy(w_hbm.at[pl.ds(0, _L)], rows, sem).wait()

        ar = lax.iota(jnp.int32, 16)
        eg0 = lax.shift_right_logical(ar, 3)
        eg1 = eg0 + 2
        s_v = lax.bitwise_and(ar, 7)

        def transpose(rows, tb):
            for l in range(_L):
                lv = jnp.full((16,), l, jnp.int32)
                plsc.store_scatter(tb, [eg0, s_v, lv], rows[l, pl.ds(0, 16)])
                plsc.store_scatter(tb, [eg1, s_v, lv], rows[l, pl.ds(16, 16)])

        def outfire(j, tb, sem):
            u = u0 + j
            hh = u // nbb
            bb = u % nbb
            pltpu.async_copy(tb.at[:, :, pl.ds(0, _L)], out_hbm.at[hh, :, bb],
                             sem)

        def owait(tb, sem):
            pltpu.make_async_copy(tb.at[:, :, pl.ds(0, _L)],
                                  out_hbm.at[0, :, 0], sem).wait()

        def halfstep(j, rows, tb, gsem, osem, rows_nxt, gsem_nxt,
                     prime, last):
            if not last:
                fire(j + 1, rows_nxt, gsem_nxt)
            drain(rows, gsem)
            if not prime:
                owait(tb, osem)
            transpose(rows, tb)
            outfire(j, tb, osem)

        fire(0, rows0, gsem0)
        halfstep(0, rows0, tb0, gsem0, osem0, rows1, gsem1, True, False)
        halfstep(1, rows1, tb1, gsem1, osem1, rows0, gsem0, True, False)

        def body(u, carry):
            a = 2 * u
            halfstep(a, rows0, tb0, gsem0, osem0, rows1, gsem1, False, False)
            halfstep(a + 1, rows1, tb1, gsem1, osem1, rows0, gsem0,
                     False, False)
            return carry

        lax.fori_loop(1, upw // 2 - 1, body, 0)

        a = upw - 2
        halfstep(a, rows0, tb0, gsem0, osem0, rows1, gsem1, False, False)
        halfstep(a + 1, rows1, tb1, gsem1, osem1, rows0, gsem0, False, True)
        owait(tb0, osem0)
        owait(tb1, osem1)

    return emb


def kernel(x, weight):
    b, h = x.shape
    v, d = weight.shape
    xr = x.T.reshape(h * (b // _L), _L).astype(jnp.int32)
    wt = jnp.pad(weight.T, ((0, 0), (0, _VP - v)))
    wt4 = wt.reshape(d // 8, 8, _VP // _L, _L).transpose(0, 2, 1, 3)
    wv = _build_tr(_VP)(wt4)
    o5 = _build(b, h)(xr, wv)
    return o5.transpose(2, 4, 0, 1, 3).reshape(b, h, d)


# speedup vs baseline: 1.1648x; 1.0346x over previous
import functools

import jax
import jax.numpy as jnp
import numpy as np
from jax import lax
from jax.experimental import pallas as pl
from jax.experimental.pallas import tpu as pltpu
from jax.experimental.pallas import tpu_sc as plsc

_L = 128
_D = 32
_W = 256
_VP = 1024000


@functools.cache
def _build_tc_tr(v):
    blk = 1024
    nblk = -(-v // blk)

    @functools.partial(
        pl.pallas_call,
        grid=(nblk,),
        in_specs=[pl.BlockSpec((_D, blk), lambda i: (0, i))],
        out_specs=pl.BlockSpec((blk * _D // _L, _L), lambda i: (i, 0)),
        out_shape=jax.ShapeDtypeStruct((nblk * blk * _D // _L, _L),
                                       jnp.float32),
    )
    def tc_tr(x_ref, o_ref):
        t = x_ref[...].T
        for k in range(_L // _D):
            o_ref[:, k * _D:(k + 1) * _D] = t[k::_L // _D]

    return tc_tr, nblk * blk


@functools.cache
def _build_tr(v):
    info = plsc.get_sparse_core_info()
    nw = info.num_cores * info.num_subcores
    nvb = v // _L
    vpw = v // nw
    n_u = vpw // _W
    uvb = _W // _L

    mesh = plsc.VectorSubcoreMesh(core_axis_name="c", subcore_axis_name="s")

    @functools.partial(
        pl.kernel,
        out_type=jax.ShapeDtypeStruct((v, _D), jnp.float32),
        mesh=mesh,
        compiler_params=pltpu.CompilerParams(
            use_tc_tiling_on_sc=False, needs_layout_passes=False),
        scratch_types=[
            pltpu.VMEM((_D // 8, uvb, 8, _L), jnp.float32),
            pltpu.VMEM((_D // 8, uvb, 8, _L), jnp.float32),
            pltpu.VMEM((_W, _D + 1), jnp.float32),
            pltpu.VMEM((_W, _D + 1), jnp.float32),
            pltpu.SemaphoreType.DMA,
            pltpu.SemaphoreType.DMA,
            pltpu.SemaphoreType.DMA,
            pltpu.SemaphoreType.DMA,
        ],
    )
    def tr(wt_hbm, out_hbm, src0, src1, tb0, tb1, ssem0, ssem1, osem0, osem1):
        wid = lax.axis_index("s") * info.num_cores + lax.axis_index("c")
        v0 = wid * vpw
        vb0 = wid * (vpw // _L)
        ar = lax.iota(jnp.int32, 16)

        def fire(u, src, sem):
            pltpu.async_copy(wt_hbm.at[:, pl.ds(vb0 + u * uvb, uvb)], src,
                             sem)

        def sdrain(src, sem):
            pltpu.make_async_copy(wt_hbm.at[:, pl.ds(0, uvb)], src,
                                  sem).wait()

        def transpose(src, tb):
            for eg in range(_D // 8):
                for s in range(8):
                    ev = jnp.full((16,), eg * 8 + s, jnp.int32)
                    for vbi in range(uvb):
                        for g in range(_L // 16):
                            lv = vbi * _L + g * 16 + ar
                            val = src[eg, vbi, s, pl.ds(g * 16, 16)]
                            plsc.store_scatter(tb, [lv, ev], val)

        def outfire(u, tb, sem):
            pltpu.async_copy(tb.at[:, pl.ds(0, _D)],
                             out_hbm.at[pl.ds(v0 + u * _W, _W)], sem)

        def owait(tb, sem):
            pltpu.make_async_copy(tb.at[:, pl.ds(0, _D)],
                                  out_hbm.at[pl.ds(v0, _W)], sem).wait()

        def halfstep(u, src, tb, ssem, osem, src_nxt, ssem_nxt, prime, last):
            if not last:
                fire(u + 1, src_nxt, ssem_nxt)
            sdrain(src, ssem)
            if not prime:
                owait(tb, osem)
            transpose(src, tb)
            outfire(u, tb, osem)

        fire(0, src0, ssem0)
        halfstep(0, src0, tb0, ssem0, osem0, src1, ssem1, True, False)
        halfstep(1, src1, tb1, ssem1, osem1, src0, ssem0, True, False)

        def body(p, carry):
            a = 2 * p + 2
            halfstep(a, src0, tb0, ssem0, osem0, src1, ssem1, False, False)
            halfstep(a + 1, src1, tb1, ssem1, osem1, src0, ssem0,
                     False, False)
            return carry

        lax.fori_loop(0, (n_u - 3) // 2, body, 0)
        halfstep(n_u - 1, src0, tb0, ssem0, osem0, src1, ssem1, False, True)
        owait(tb0, osem0)
        owait(tb1, osem1)

    return tr


@functools.cache
def _build(b, h):
    info = plsc.get_sparse_core_info()
    nw = info.num_cores * info.num_subcores
    nbb = b // _L
    n_units = h * nbb
    upw = n_units // nw

    mesh = plsc.VectorSubcoreMesh(core_axis_name="c", subcore_axis_name="s")

    @functools.partial(
        pl.kernel,
        out_type=jax.ShapeDtypeStruct((h, _D // 8, nbb, 8, _L), jnp.float32),
        mesh=mesh,
        compiler_params=pltpu.CompilerParams(
            use_tc_tiling_on_sc=False, needs_layout_passes=False),
        scratch_types=[
            pltpu.VMEM((upw, _L), jnp.int32),
            pltpu.VMEM((_L, _L), jnp.float32),
            pltpu.VMEM((_L, _L), jnp.float32),
            pltpu.VMEM((_D // 8, 8, _L + 1), jnp.float32),
            pltpu.VMEM((_D // 8, 8, _L + 1), jnp.float32),
            pltpu.SemaphoreType.DMA,
            pltpu.SemaphoreType.DMA,
            pltpu.SemaphoreType.DMA,
            pltpu.SemaphoreType.DMA,
        ],
    )
    def emb(x_hbm, w_hbm, out_hbm, idx_v, rows0, rows1,
            tb0, tb1, gsem0, gsem1, osem0, osem1):
        wid = lax.axis_index("s") * info.num_cores + lax.axis_index("c")
        u0 = wid * upw
        pltpu.sync_copy(x_hbm.at[pl.ds(u0, upw)], idx_v)

        def fire(j, rows, sem):
            pltpu.async_copy(w_hbm.at[idx_v.at[j]], rows, sem)

        def drain(rows, sem):
            pltpu.make_async_copy(w_hbm.at[pl.ds(0, _L)], rows, sem).wait()

        ar = lax.iota(jnp.int32, 16)
        eg0 = lax.shift_right_logical(ar, 3)
        eg1 = eg0 + 2
        s_v = lax.bitwise_and(ar, 7)

        def transpose(rows, tb):
            for l in range(_L):
                lv = jnp.full((16,), l, jnp.int32)
                plsc.store_scatter(tb, [eg0, s_v, lv], rows[l, pl.ds(0, 16)])
                plsc.store_scatter(tb, [eg1, s_v, lv], rows[l, pl.ds(16, 16)])

        def outfire(j, tb, sem):
            u = u0 + j
            hh = u // nbb
            bb = u % nbb
            pltpu.async_copy(tb.at[:, :, pl.ds(0, _L)], out_hbm.at[hh, :, bb],
                             sem)

        def owait(tb, sem):
            pltpu.make_async_copy(tb.at[:, :, pl.ds(0, _L)],
                                  out_hbm.at[0, :, 0], sem).wait()

        def halfstep(j, rows, tb, gsem, osem, rows_nxt, gsem_nxt,
                     prime, last):
            if not last:
                fire(j + 1, rows_nxt, gsem_nxt)
            drain(rows, gsem)
            if not prime:
                owait(tb, osem)
            transpose(rows, tb)
            outfire(j, tb, osem)

        fire(0, rows0, gsem0)
        halfstep(0, rows0, tb0, gsem0, osem0, rows1, gsem1, True, False)
        halfstep(1, rows1, tb1, gsem1, osem1, rows0, gsem0, True, False)

        def body(u, carry):
            a = 2 * u
            halfstep(a, rows0, tb0, gsem0, osem0, rows1, gsem1, False, False)
            halfstep(a + 1, rows1, tb1, gsem1, osem1, rows0, gsem0,
                     False, False)
            return carry

        lax.fori_loop(1, upw // 2 - 1, body, 0)

        a = upw - 2
        halfstep(a, rows0, tb0, gsem0, osem0, rows1, gsem1, False, False)
        halfstep(a + 1, rows1, tb1, gsem1, osem1, rows0, gsem0, False, True)
        owait(tb0, osem0)
        owait(tb1, osem1)

    return emb


def kernel(x, weight):
    b, h = x.shape
    v, d = weight.shape
    xr = x.T.reshape(h * (b // _L), _L).astype(jnp.int32)
    wp = jnp.pad(weight, ((0, 0), (0, _L - d)))
    o5 = _build(b, h)(xr, wp)
    return o5.transpose(2, 4, 0, 1, 3).reshape(b, h, d)


# speedup vs baseline: 1.2328x; 1.0584x over previous
import functools

import jax
import jax.numpy as jnp
from jax import lax
from jax.experimental import pallas as pl
from jax.experimental.pallas import tpu as pltpu
from jax.experimental.pallas import tpu_sc as plsc

_L = 128
_D = 32


@functools.cache
def _build(b, h):
    info = plsc.get_sparse_core_info()
    nw = info.num_cores * info.num_subcores
    nbb = b // _L
    n_units = h * nbb
    upw = n_units // nw

    mesh = plsc.VectorSubcoreMesh(core_axis_name="c", subcore_axis_name="s")

    @functools.partial(
        pl.kernel,
        out_type=jax.ShapeDtypeStruct((h, _D // 8, nbb, 8, _L), jnp.float32),
        mesh=mesh,
        compiler_params=pltpu.CompilerParams(
            use_tc_tiling_on_sc=False, needs_layout_passes=False),
        scratch_types=[
            pltpu.VMEM((upw, _L), jnp.int32),
            pltpu.VMEM((_L, _D), jnp.float32),
            pltpu.VMEM((_L, _D), jnp.float32),
            pltpu.VMEM((_D // 8, 8, _L + 1), jnp.float32),
            pltpu.VMEM((_D // 8, 8, _L + 1), jnp.float32),
            pltpu.SemaphoreType.DMA,
            pltpu.SemaphoreType.DMA,
            pltpu.SemaphoreType.DMA,
            pltpu.SemaphoreType.DMA,
        ],
    )
    def emb(x_hbm, w_hbm, out_hbm, idx_v, rows0, rows1,
            tb0, tb1, gsem0, gsem1, osem0, osem1):
        wid = lax.axis_index("s") * info.num_cores + lax.axis_index("c")
        u0 = wid * upw
        pltpu.sync_copy(x_hbm.at[pl.ds(u0, upw)], idx_v)

        def fire(j, rows, sem):
            pltpu.async_copy(w_hbm.at[idx_v.at[j]], rows, sem)

        def drain(rows, sem):
            pltpu.make_async_copy(w_hbm.at[pl.ds(0, _L)], rows, sem).wait()

        ar = lax.iota(jnp.int32, 16)
        eg0 = lax.shift_right_logical(ar, 3)
        eg1 = eg0 + 2
        s_v = lax.bitwise_and(ar, 7)

        def transpose(rows, tb):
            for l in range(_L):
                lv = jnp.full((16,), l, jnp.int32)
                plsc.store_scatter(tb, [eg0, s_v, lv], rows[l, pl.ds(0, 16)])
                plsc.store_scatter(tb, [eg1, s_v, lv], rows[l, pl.ds(16, 16)])

        def outfire(j, tb, sem):
            u = u0 + j
            hh = u // nbb
            bb = u % nbb
            pltpu.async_copy(tb.at[:, :, pl.ds(0, _L)], out_hbm.at[hh, :, bb],
                             sem)

        def owait(tb, sem):
            pltpu.make_async_copy(tb.at[:, :, pl.ds(0, _L)],
                                  out_hbm.at[0, :, 0], sem).wait()

        def halfstep(j, rows, tb, gsem, osem, rows_nxt, gsem_nxt,
                     prime, last):
            if not last:
                fire(j + 1, rows_nxt, gsem_nxt)
            drain(rows, gsem)
            if not prime:
                owait(tb, osem)
            transpose(rows, tb)
            outfire(j, tb, osem)

        fire(0, rows0, gsem0)
        halfstep(0, rows0, tb0, gsem0, osem0, rows1, gsem1, True, False)
        halfstep(1, rows1, tb1, gsem1, osem1, rows0, gsem0, True, False)

        def body(u, carry):
            a = 2 * u
            halfstep(a, rows0, tb0, gsem0, osem0, rows1, gsem1, False, False)
            halfstep(a + 1, rows1, tb1, gsem1, osem1, rows0, gsem0,
                     False, False)
            return carry

        lax.fori_loop(1, upw // 2 - 1, body, 0)

        a = upw - 2
        halfstep(a, rows0, tb0, gsem0, osem0, rows1, gsem1, False, False)
        halfstep(a + 1, rows1, tb1, gsem1, osem1, rows0, gsem0, False, True)
        owait(tb0, osem0)
        owait(tb1, osem1)

    return emb


def kernel(x, weight):
    b, h = x.shape
    _, d = weight.shape
    xr = x.T.reshape(h * (b // _L), _L).astype(jnp.int32)
    o5 = _build(b, h)(xr, weight)
    return o5.transpose(2, 4, 0, 1, 3).reshape(b, h, d)
